# Initial kernel scaffold; baseline (speedup 1.0000x reference)
import jax
import jax.numpy as jnp
from jax.experimental import pallas as pl


def kernel(beta, x, object_id, weights):
    raise NotImplementedError("write your pallas kernel here")


# speedup vs baseline: 1.0414x; 1.0414x over previous
import functools

import jax
import jax.numpy as jnp
from jax import lax
from jax.experimental import pallas as pl
from jax.experimental.pallas import tpu as pltpu

Q_MIN = 0.1
EPS = 1e-09
N = 50000
D = 8
KP = 512
B = 1000
TB = N // B
BIG_I32 = 2**30


def _phase1_body(beta_ref, oid_ref, q_ref, stats_ref):
    def tile(t, carry):
        gmax, gidx, cnt, nb, nc = carry
        b = beta_ref[t]
        q = 0.5 * jnp.log((1.0 + b) / (1.0 - b))
        q = q * q + Q_MIN
        q_ref[t] = q
        ids = oid_ref[t]
        ids2 = ids[:, None]
        q2 = q[:, None]
        cand = lax.broadcasted_iota(jnp.int32, (B, KP), 1) + 1
        rows = lax.broadcasted_iota(jnp.int32, (B, KP), 0) + t * B
        m = ids2 == cand
        cnt = cnt + jnp.sum(m.astype(jnp.float32), axis=0, keepdims=True)
        qm = jnp.where(m, q2, -1.0)
        tmax = jnp.max(qm, axis=0, keepdims=True)
        eq = (qm == tmax) & m
        tidx = jnp.min(jnp.where(eq, rows, BIG_I32), axis=0, keepdims=True)
        upd = tmax > gmax
        gmax = jnp.where(upd, tmax, gmax)
        gidx = jnp.where(upd, tidx, gidx)
        noise = ids2 == 0
        nb = nb + jnp.sum(jnp.where(noise, b[:, None], 0.0))
        nc = nc + jnp.sum(noise.astype(jnp.float32))
        return gmax, gidx, cnt, nb, nc

    init = (
        jnp.full((1, KP), -1.0, jnp.float32),
        jnp.zeros((1, KP), jnp.int32),
        jnp.zeros((1, KP), jnp.float32),
        jnp.float32(0.0),
        jnp.float32(0.0),
    )
    gmax, gidx, cnt, nb, nc = lax.fori_loop(0, TB, tile, init)
    stats_ref[0:1, :] = gidx.astype(jnp.float32)
    stats_ref[1:2, :] = cnt
    stats_ref[2:3, :] = gmax
    stats_ref[3:4, :] = jnp.full((1, KP), nb, jnp.float32)
    stats_ref[4:5, :] = jnp.full((1, KP), nc, jnp.float32)
    stats_ref[5:8, :] = jnp.zeros((3, KP), jnp.float32)


def _phase2_body(alpha_ref, xf_ref, oid_ref, q_ref, w_ref, stats_ref,
                 out_ref, xk_ref):
    def gather(k, _):
        a = alpha_ref[k]
        xk_ref[pl.ds(k, 1), :] = xf_ref[pl.ds(a, 1), :]
        return 0
    lax.fori_loop(0, KP, gather, 0)

    cnt = stats_ref[1:2, :]
    qmax = stats_ref[2:3, :]
    nb = stats_ref[3, 0]
    nc = stats_ref[4, 0]

    present = cnt > 0.0
    k_f = jnp.sum(present.astype(jnp.float32))
    qk = jnp.where(present, qmax, 0.0)
    c_att = qk / ((cnt + EPS) * k_f)
    c_rep = qk / ((jnp.float32(N) - cnt + EPS) * k_f)

    xk = xk_ref[...]
    xkxk = jnp.sum(xk * xk, axis=1)[None, :]

    def tile(t, carry):
        s_rep, s_att = carry
        xt = xf_ref[pl.ds(t * B, B), :]
        ids = oid_ref[t][:, None]
        a = (w_ref[t] * q_ref[t])[:, None]
        xx = jnp.sum(xt * xt, axis=1)[:, None]
        cross = lax.dot_general(
            xt, xk, (((1,), (1,)), ((), ())),
            preferred_element_type=jnp.float32,
            precision=lax.Precision.HIGHEST)
        d2 = jnp.maximum(xx + xkxk - 2.0 * cross, 0.0)
        dist = jnp.sqrt(d2 + 1e-12)
        cand = lax.broadcasted_iota(jnp.int32, (B, KP), 1) + 1
        attm = ids == cand
        rep_e = jnp.where(attm, 0.0, jnp.maximum(1.0 - dist, 0.0))
        att_e = jnp.where(attm, d2, 0.0)
        s_rep = s_rep + jnp.sum(a * rep_e, axis=0, keepdims=True)
        s_att = s_att + jnp.sum(a * att_e, axis=0, keepdims=True)
        return s_rep, s_att

    z = jnp.zeros((1, KP), jnp.float32)
    s_rep, s_att = lax.fori_loop(0, TB, tile, (z, z))

    v_att = jnp.sum(s_att * c_att)
    v_rep = jnp.sum(s_rep * c_rep)
    beta_k = jnp.tanh(jnp.sqrt(jnp.maximum(qmax - Q_MIN, 0.0)))
    l_cow = jnp.sum(jnp.where(present, 1.0 - beta_k, 0.0)) / k_f
    l_noise = nb / jnp.maximum(nc, 1.0)

    li = lax.broadcasted_iota(jnp.int32, (8, 128), 1)
    out = jnp.where(li == 0, v_att,
          jnp.where(li == 1, v_rep,
          jnp.where(li == 2, l_cow,
          jnp.where(li == 3, l_noise, 0.0))))
    out_ref[...] = out


@jax.jit
def kernel(beta, x, object_id, weights):
    beta2 = beta.reshape(TB, B)
    oid2 = object_id.reshape(TB, B)
    w2 = weights.reshape(TB, B)

    q2, stats = pl.pallas_call(
        _phase1_body,
        out_shape=(
            jax.ShapeDtypeStruct((TB, B), jnp.float32),
            jax.ShapeDtypeStruct((8, KP), jnp.float32),
        ),
    )(beta2, oid2)

    alphas = stats[0].astype(jnp.int32)

    out = pl.pallas_call(
        _phase2_body,
        out_shape=jax.ShapeDtypeStruct((8, 128), jnp.float32),
        in_specs=[
            pl.BlockSpec(memory_space=pltpu.MemorySpace.SMEM),
            pl.BlockSpec(memory_space=pltpu.MemorySpace.VMEM),
            pl.BlockSpec(memory_space=pltpu.MemorySpace.VMEM),
            pl.BlockSpec(memory_space=pltpu.MemorySpace.VMEM),
            pl.BlockSpec(memory_space=pltpu.MemorySpace.VMEM),
            pl.BlockSpec(memory_space=pltpu.MemorySpace.VMEM),
        ],
        scratch_shapes=[pltpu.VMEM((KP, D), jnp.float32)],
    )(alphas, x, oid2, q2, w2, stats)

    return (out[0, 0], out[0, 1], out[0, 2], out[0, 3])


# speedup vs baseline: 1.5355x; 1.4744x over previous
import functools

import jax
import jax.numpy as jnp
from jax import lax
from jax.experimental import pallas as pl
from jax.experimental.pallas import tpu as pltpu

Q_MIN = 0.1
EPS = 1e-09
N = 50000
D = 8
KP = 512
B = 2000
TB = N // B
BIG_I32 = 2**30


def _phase1_body(beta_ref, oid_ref, w_ref, a_ref, stats_ref):
    cand = lax.broadcasted_iota(jnp.int32, (B, KP), 1) + 1
    rows = lax.broadcasted_iota(jnp.int32, (B, KP), 0)

    def tile(t, carry):
        gmax, gidx, cnt, nb, nc = carry
        b = beta_ref[t]
        q = 0.5 * jnp.log((1.0 + b) / (1.0 - b))
        q = q * q + Q_MIN
        a_ref[t] = w_ref[t] * q
        ids = oid_ref[t]
        ids2 = ids[:, None]
        q2 = q[:, None]
        m = ids2 == cand
        cnt = cnt + jnp.sum(m.astype(jnp.float32), axis=0, keepdims=True)
        qm = jnp.where(m, q2, -1.0)
        tmax = jnp.max(qm, axis=0, keepdims=True)
        tidx = jnp.min(jnp.where(qm == tmax, rows, BIG_I32),
                       axis=0, keepdims=True)
        upd = tmax > gmax
        gmax = jnp.where(upd, tmax, gmax)
        gidx = jnp.where(upd, tidx + t * B, gidx)
        noise = ids2 == 0
        nb = nb + jnp.sum(jnp.where(noise, b[:, None], 0.0))
        nc = nc + jnp.sum(noise.astype(jnp.float32))
        return gmax, gidx, cnt, nb, nc

    init = (
        jnp.full((1, KP), -1.0, jnp.float32),
        jnp.zeros((1, KP), jnp.int32),
        jnp.zeros((1, KP), jnp.float32),
        jnp.float32(0.0),
        jnp.float32(0.0),
    )
    gmax, gidx, cnt, nb, nc = lax.fori_loop(0, TB, tile, init)
    stats_ref[0:1, :] = gidx.astype(jnp.float32)
    stats_ref[1:2, :] = cnt
    stats_ref[2:3, :] = gmax
    stats_ref[3:4, :] = jnp.full((1, KP), nb, jnp.float32)
    stats_ref[4:5, :] = jnp.full((1, KP), nc, jnp.float32)
    stats_ref[5:8, :] = jnp.zeros((3, KP), jnp.float32)


def _phase2_body(alpha_ref, xf_ref, oid_ref, a_ref, stats_ref,
                 out_ref, xk_ref):
    def gather(k, _):
        a = alpha_ref[k]
        xk_ref[pl.ds(k, 1), :] = xf_ref[pl.ds(a, 1), :]
        return 0
    lax.fori_loop(0, KP, gather, 0)

    cnt = stats_ref[1:2, :]
    qmax = stats_ref[2:3, :]
    nb = stats_ref[3, 0]
    nc = stats_ref[4, 0]

    present = cnt > 0.0
    k_f = jnp.sum(present.astype(jnp.float32))
    qk = jnp.where(present, qmax, 0.0)
    c_att = qk / ((cnt + EPS) * k_f)
    c_rep = qk / ((jnp.float32(N) - cnt + EPS) * k_f)

    xk = xk_ref[...]
    xkT = xk.T
    xkxk = jnp.sum(xkT * xkT, axis=0, keepdims=True)
    cand = lax.broadcasted_iota(jnp.int32, (B, KP), 1) + 1

    def tile(t, carry):
        s_rep, s_att = carry
        xt = xf_ref[pl.ds(t * B, B), :]
        ids = oid_ref[t][:, None]
        a = a_ref[t][:, None]
        xx = jnp.sum(xt * xt, axis=1)[:, None]
        cross = lax.dot_general(
            xt, xkT, (((1,), (0,)), ((), ())),
            preferred_element_type=jnp.float32)
        d2 = jnp.maximum(xx + xkxk - 2.0 * cross, 0.0)
        dist = jnp.sqrt(d2 + 1e-12)
        attm = ids == cand
        rep_e = jnp.where(attm, 0.0, jnp.maximum(1.0 - dist, 0.0))
        att_e = jnp.where(attm, d2, 0.0)
        s_rep = s_rep + jnp.sum(a * rep_e, axis=0, keepdims=True)
        s_att = s_att + jnp.sum(a * att_e, axis=0, keepdims=True)
        return s_rep, s_att

    z = jnp.zeros((1, KP), jnp.float32)
    s_rep, s_att = lax.fori_loop(0, TB, tile, (z, z))

    v_att = jnp.sum(s_att * c_att)
    v_rep = jnp.sum(s_rep * c_rep)
    beta_k = jnp.tanh(jnp.sqrt(jnp.maximum(qmax - Q_MIN, 0.0)))
    l_cow = jnp.sum(jnp.where(present, 1.0 - beta_k, 0.0)) / k_f
    l_noise = nb / jnp.maximum(nc, 1.0)

    li = lax.broadcasted_iota(jnp.int32, (8, 128), 1)
    out = jnp.where(li == 0, v_att,
          jnp.where(li == 1, v_rep,
          jnp.where(li == 2, l_cow,
          jnp.where(li == 3, l_noise, 0.0))))
    out_ref[...] = out


@jax.jit
def kernel(beta, x, object_id, weights):
    beta2 = beta.reshape(TB, B)
    oid2 = object_id.reshape(TB, B)
    w2 = weights.reshape(TB, B)

    a2, stats = pl.pallas_call(
        _phase1_body,
        out_shape=(
            jax.ShapeDtypeStruct((TB, B), jnp.float32),
            jax.ShapeDtypeStruct((8, KP), jnp.float32),
        ),
    )(beta2, oid2, w2)

    alphas = stats[0].astype(jnp.int32)

    out = pl.pallas_call(
        _phase2_body,
        out_shape=jax.ShapeDtypeStruct((8, 128), jnp.float32),
        in_specs=[
            pl.BlockSpec(memory_space=pltpu.MemorySpace.SMEM),
            pl.BlockSpec(memory_space=pltpu.MemorySpace.VMEM),
            pl.BlockSpec(memory_space=pltpu.MemorySpace.VMEM),
            pl.BlockSpec(memory_space=pltpu.MemorySpace.VMEM),
            pl.BlockSpec(memory_space=pltpu.MemorySpace.VMEM),
        ],
        scratch_shapes=[pltpu.VMEM((KP, D), jnp.float32)],
    )(alphas, x, oid2, a2, stats)

    return (out[0, 0], out[0, 1], out[0, 2], out[0, 3])
